# baseline (speedup 1.0000x reference)
import functools

import jax
import jax.numpy as jnp
from jax import lax
from jax.experimental import pallas as pl
from jax.experimental.pallas import tpu as pltpu
from jax.experimental.pallas import tpu_sc as plsc

B, N, T = 1024, 26, 20
M = B * N * T
D = 16
V = 100000
F = M * 3
NC, NS = 2, 16
NW = NC * NS
CHUNK = F // NW
SUB = 3328
N_ITERS = CHUNK // SUB

_mesh = plsc.VectorSubcoreMesh(core_axis_name="c", subcore_axis_name="s")


@functools.partial(
    pl.kernel,
    mesh=_mesh,
    compiler_params=pltpu.CompilerParams(use_tc_tiling_on_sc=False),
    out_type=jax.ShapeDtypeStruct((F, D), jnp.float32),
    scratch_types=[
        [pltpu.VMEM((SUB,), jnp.int32)] * 2,
        [pltpu.VMEM((SUB, D), jnp.float32)] * 2,
        [pltpu.SemaphoreType.DMA] * 2,
        [pltpu.SemaphoreType.DMA] * 2,
    ],
)
def _embed(idx_hbm, table, out, idx_v, rows_v, sem_g, sem_w):
    wid = lax.axis_index("s") * NC + lax.axis_index("c")
    base = wid * CHUNK

    gathers = {}
    writes = {}
    for j in range(N_ITERS):
        s = j % 2
        start = base + j * SUB
        if j >= 2:
            writes[j - 2].wait()
        pltpu.sync_copy(idx_hbm.at[pl.ds(start, SUB)], idx_v[s])
        gathers[j] = pltpu.async_copy(table.at[idx_v[s]], rows_v[s], sem_g[s])
        if j >= 1:
            gathers[j - 1].wait()
            writes[j - 1] = pltpu.async_copy(
                rows_v[1 - s], out.at[pl.ds(start - SUB, SUB)], sem_w[1 - s]
            )
    gathers[N_ITERS - 1].wait()
    last = base + (N_ITERS - 1) * SUB
    writes[N_ITERS - 1] = pltpu.async_copy(
        rows_v[(N_ITERS - 1) % 2], out.at[pl.ds(last, SUB)], sem_w[(N_ITERS - 1) % 2]
    )
    writes[N_ITERS - 2].wait()
    writes[N_ITERS - 1].wait()


def kernel(x, W0, W1, W2):
    table = jnp.concatenate([W0, W1, W2], axis=0)
    idx = (x + jnp.arange(3, dtype=jnp.int32) * V).reshape(F)
    out = _embed(idx, table)
    return out.reshape(B, N, T, 3 * D)


# speedup vs baseline: 1.0032x; 1.0032x over previous
import functools

import jax
import jax.numpy as jnp
from jax import lax
from jax.experimental import pallas as pl
from jax.experimental.pallas import tpu as pltpu
from jax.experimental.pallas import tpu_sc as plsc

B, N, T = 1024, 26, 20
M = B * N * T
D = 16
F = M * 3
NC, NS = 2, 16
NW = NC * NS
CHUNK = F // NW
SUB = 3120
SUB3 = SUB // 3
VECS = SUB // 48
N_ITERS = CHUNK // SUB

_mesh = plsc.VectorSubcoreMesh(core_axis_name="c", subcore_axis_name="s")


@functools.partial(
    pl.kernel,
    mesh=_mesh,
    compiler_params=pltpu.CompilerParams(
        use_tc_tiling_on_sc=False, needs_layout_passes=False
    ),
    out_type=jax.ShapeDtypeStruct((M, 3 * D), jnp.float32),
    scratch_types=[
        [pltpu.VMEM((SUB,), jnp.int32)] * 2,
        [[pltpu.VMEM((SUB3,), jnp.int32)] * 3] * 2,
        [[pltpu.VMEM((SUB3, D), jnp.float32)] * 3] * 2,
        [pltpu.SemaphoreType.DMA] * 2,
        [pltpu.SemaphoreType.DMA] * 2,
    ],
)
def _embed(xflat, w0, w1, w2, out, idx3_v, idxt_v, rows_v, sem_g, sem_w):
    wid = lax.axis_index("s") * NC + lax.axis_index("c")
    base = wid * CHUNK
    tables = (w0, w1, w2)
    lanes3 = lax.iota(jnp.int32, 16) * 3

    gathers = {}
    writes = {}

    def start_chunk(j, s):
        f0 = base + j * SUB
        pltpu.sync_copy(xflat.at[pl.ds(f0, SUB)], idx3_v[s])

        def deint(m, _):
            off = m * 48
            for i in range(3):
                vec = plsc.load_gather(idx3_v[s], [lanes3 + (off + i)])
                idxt_v[s][i][pl.ds(m * 16, 16)] = vec
            return 0

        lax.fori_loop(0, VECS, deint, 0)
        for i in range(3):
            gathers[(j, i)] = pltpu.async_copy(
                tables[i].at[idxt_v[s][i]], rows_v[s][i], sem_g[s]
            )

    def write_chunk(j, s):
        p0 = (base + j * SUB) // 3
        for i in range(3):
            gathers[(j, i)].wait()
        for i in range(3):
            writes[(j, i)] = pltpu.async_copy(
                rows_v[s][i], out.at[pl.ds(p0, SUB3), pl.ds(i * D, D)], sem_w[s]
            )

    for j in range(N_ITERS):
        s = j % 2
        if j >= 2:
            for i in range(3):
                writes[(j - 2, i)].wait()
        start_chunk(j, s)
        if j >= 1:
            write_chunk(j - 1, 1 - s)
    write_chunk(N_ITERS - 1, (N_ITERS - 1) % 2)
    for j in (N_ITERS - 2, N_ITERS - 1):
        for i in range(3):
            writes[(j, i)].wait()


def kernel(x, W0, W1, W2):
    out = _embed(x.reshape(F), W0, W1, W2)
    return out.reshape(B, N, T, 3 * D)


# speedup vs baseline: 3.5615x; 3.5501x over previous
import functools

import jax
import jax.numpy as jnp
from jax import lax
from jax.experimental import pallas as pl
from jax.experimental.pallas import tpu as pltpu
from jax.experimental.pallas import tpu_sc as plsc

B, N, T = 1024, 26, 20
M = B * N * T
D = 16
NC, NS = 2, 16
NW = NC * NS
CHUNK = M // NW
SUB = 1040
N_ITERS = CHUNK // SUB

_mesh = plsc.VectorSubcoreMesh(core_axis_name="c", subcore_axis_name="s")


@functools.partial(
    pl.kernel,
    mesh=_mesh,
    compiler_params=pltpu.CompilerParams(use_tc_tiling_on_sc=False),
    out_type=jax.ShapeDtypeStruct((M, 3 * D), jnp.float32),
    scratch_types=[
        [[pltpu.VMEM((SUB,), jnp.int32)] * 3] * 2,
        [[pltpu.VMEM((SUB, D), jnp.float32)] * 3] * 2,
        [pltpu.SemaphoreType.DMA] * 2,
        [pltpu.SemaphoreType.DMA] * 2,
        [pltpu.SemaphoreType.DMA] * 2,
    ],
)
def _embed(xt, w0, w1, w2, out, idx_v, rows_v, sem_i, sem_g, sem_w):
    wid = lax.axis_index("s") * NC + lax.axis_index("c")
    base = wid * CHUNK
    tables = (w0, w1, w2)

    idx_cps = {}
    gathers = {}
    writes = {}

    def fetch_idx(j, s):
        p0 = base + j * SUB
        for i in range(3):
            idx_cps[(j, i)] = pltpu.async_copy(
                xt.at[i, pl.ds(p0, SUB)], idx_v[s][i], sem_i[s]
            )

    def start_gathers(j, s):
        for i in range(3):
            idx_cps[(j, i)].wait()
        for i in range(3):
            gathers[(j, i)] = pltpu.async_copy(
                tables[i].at[idx_v[s][i]], rows_v[s][i], sem_g[s]
            )

    def write_out(j, s):
        p0 = base + j * SUB
        for i in range(3):
            gathers[(j, i)].wait()
        for i in range(3):
            writes[(j, i)] = pltpu.async_copy(
                rows_v[s][i], out.at[pl.ds(p0, SUB), pl.ds(i * D, D)], sem_w[s]
            )

    fetch_idx(0, 0)
    for j in range(N_ITERS):
        s = j % 2
        if j >= 2:
            for i in range(3):
                writes[(j - 2, i)].wait()
        start_gathers(j, s)
        if j >= 1:
            write_out(j - 1, 1 - s)
        if j + 1 < N_ITERS:
            fetch_idx(j + 1, 1 - s)
    write_out(N_ITERS - 1, (N_ITERS - 1) % 2)
    for j in (N_ITERS - 2, N_ITERS - 1):
        for i in range(3):
            writes[(j, i)].wait()


def kernel(x, W0, W1, W2):
    xt = x.reshape(M, 3).T
    out = _embed(xt, W0, W1, W2)
    return out.reshape(B, N, T, 3 * D)
